# baseline (speedup 1.0000x reference)
import functools
import math

import jax
import jax.numpy as jnp
from jax import lax
from jax.experimental import pallas as pl
from jax.experimental.pallas import tpu as pltpu
from jax.experimental.pallas import tpu_sc as plsc

N = 10000
E = 320000
D = 128
NC = 2
NS = 16
NW = NC * NS
EPW = E // NW
B = 125
NCHUNK = EPW // B
NPAD = 10112
TPR = NPAD // NS


_MESH = plsc.VectorSubcoreMesh(core_axis_name="c", subcore_axis_name="s")


def _sc_segment_sums(src, dst, x, h0, h0sq, h0cub, zrow, ones):
    @functools.partial(
        pl.kernel,
        mesh=_MESH,
        out_type=jax.ShapeDtypeStruct((5, NC, NPAD, D), jnp.float32),
        scratch_types=[
            pltpu.VMEM((B,), jnp.int32),
            pltpu.VMEM((B,), jnp.int32),
            pltpu.VMEM((B,), jnp.int32),
            pltpu.VMEM((B,), jnp.int32),
            pltpu.VMEM((B,), jnp.int32),
            pltpu.VMEM((B,), jnp.int32),
            pltpu.VMEM((B, D), jnp.float32),
            pltpu.VMEM((B, D), jnp.float32),
            pltpu.VMEM((B, D), jnp.float32),
            pltpu.VMEM_SHARED((NPAD, D), jnp.float32),
        ] + [pltpu.SemaphoreType.DMA] * 12,
    )
    def k(src_h, dst_h, x_h, h0_h, h0sq_h, h0cub_h, zrow_h, ones_h,
          acc_out, src0_v, src1_v, src2_v, dst0_v, dst1_v, dst2_v,
          rows0_v, rows1_v, rows2_v, acc_sh, gsem0, gsem1, gsem2,
          ssem0, ssem1, ssem2, dsem0, dsem1, dsem2, sisem0, sisem1,
          sisem2):
        core = lax.axis_index("c")
        tile = lax.axis_index("s")
        wid = core * NS + tile
        rbase = pl.multiple_of(tile * TPR, 8)

        rows = (rows0_v, rows1_v, rows2_v)
        srcs = (src0_v, src1_v, src2_v)
        dsts = (dst0_v, dst1_v, dst2_v)
        gsems = (gsem0, gsem1, gsem2)
        ssems = (ssem0, ssem1, ssem2)
        dsems = (dsem0, dsem1, dsem2)
        sisems = (sisem0, sisem1, sisem2)

        for p, tab in enumerate((x_h, h0_h, h0sq_h, h0cub_h, None)):
            pltpu.sync_copy(zrow_h, acc_sh.at[pl.ds(rbase, TPR)])
            if tab is None:
                pltpu.sync_copy(ones_h, rows2_v)
            plsc.subcore_barrier()

            pltpu.async_copy(dst_h.at[wid, 0], dst0_v, dsem0)
            pltpu.async_copy(dst_h.at[wid, 1], dst1_v, dsem1)
            if tab is not None:
                pltpu.sync_copy(src_h.at[wid, 0], src0_v)
                pltpu.sync_copy(src_h.at[wid, 1], src1_v)
                pltpu.async_copy(src_h.at[wid, 2], src2_v, sisem2)
                pltpu.async_copy(tab.at[src0_v], rows0_v, gsem0)
                pltpu.async_copy(tab.at[src1_v], rows1_v, gsem1)

            @pl.loop(0, NCHUNK + 1, step=3)
            def _chunk(i):
                for b in (0, 1, 2):
                    c = i + b
                    bp = (b - 1) % 3
                    cc = jnp.minimum(c, NCHUNK - 1)
                    c2 = jnp.minimum(c + 2, NCHUNK - 1)
                    c3 = jnp.minimum(c + 3, NCHUNK - 1)
                    srows = rows2_v if tab is None else rows[b]

                    @pl.when(c < NCHUNK)
                    def _():
                        pltpu.make_async_copy(dst_h.at[wid, cc],
                                              dsts[b], dsems[b]).wait()
                        if tab is not None:
                            pltpu.make_async_copy(tab.at[srcs[b]],
                                                  rows[b], gsems[b]).wait()

                            @pl.when(c + 3 < NCHUNK)
                            def _():
                                pltpu.async_copy(src_h.at[wid, c3],
                                                 srcs[b], sisems[b])

                        pltpu.async_copy(srows, acc_sh.at[dsts[b]],
                                         ssems[b], add=True)

                        @pl.when(c >= 1)
                        def _():
                            sprev = rows2_v if tab is None else rows[bp]
                            pltpu.make_async_copy(
                                sprev, acc_sh.at[dsts[bp]],
                                ssems[bp]).wait()

                        @pl.when(c + 2 < NCHUNK)
                        def _():
                            pltpu.async_copy(dst_h.at[wid, c2],
                                             dsts[bp], dsems[bp])
                            if tab is not None:
                                pltpu.make_async_copy(
                                    src_h.at[wid, c2], srcs[bp],
                                    sisems[bp]).wait()
                                pltpu.async_copy(tab.at[srcs[bp]],
                                                 rows[bp], gsems[bp])

            fb = (NCHUNK - 1) % 3
            fsrows = rows2_v if tab is None else rows[fb]
            pltpu.make_async_copy(fsrows, acc_sh.at[dsts[fb]],
                                  ssems[fb]).wait()

            plsc.subcore_barrier()
            pltpu.sync_copy(acc_sh.at[pl.ds(rbase, TPR)],
                            acc_out.at[p, core, pl.ds(rbase, TPR)])

    return k(src, dst, x, h0, h0sq, h0cub, zrow, ones)


def _powers_body(h0_ref, sq_ref, cub_ref):
    h = h0_ref[...]
    s = h * h
    sq_ref[...] = s
    cub_ref[...] = s * h


def _epilogue_body(scal_ref, acc_ref, h0_ref, W_ref, Watt_ref, out_ref):
    theta = scal_ref[0]
    alpha = scal_ref[1]
    beta = 0.1

    deg = acc_ref[4, 0, :, :1] + acc_ref[4, 1, :, :1]
    invd = 1.0 / jnp.maximum(deg, 1.0)

    hx = (acc_ref[0, 0] + acc_ref[0, 1]) * invd
    hm = (acc_ref[1, 0] + acc_ref[1, 1]) * invd
    hs = (acc_ref[2, 0] + acc_ref[2, 1]) * invd
    hg = (acc_ref[3, 0] + acc_ref[3, 1]) * invd

    h_agg = (1.0 - alpha) * hx + alpha * h0_ref[...]
    hW = jnp.dot(h_agg, W_ref[...], preferred_element_type=jnp.float32)
    h_i = theta * hW + (1.0 - theta) * h_agg

    mu = hm
    s = jnp.where(hs == 0.0, 1e-16, hs)
    sigma = jnp.sqrt(s)
    g = jnp.where(hg == 0.0, 1e-16, hg)
    neg = g < 0.0
    ga = jnp.abs(g)
    gr = jnp.exp(jnp.log(ga) * (1.0 / 3.0))
    gamma = jnp.where(neg, -gr, gr)

    Wk = Watt_ref[:D, :]
    Wq = Watt_ref[D:, :]
    q = jnp.dot(h_i, Wq, preferred_element_type=jnp.float32)

    def elu(v):
        return jnp.where(v > 0.0, v, jnp.exp(jnp.minimum(v, 0.0)) - 1.0)

    e0 = elu(jnp.dot(mu, Wk, preferred_element_type=jnp.float32) + q)
    e1 = elu(jnp.dot(sigma, Wk, preferred_element_type=jnp.float32) + q)
    e2 = elu(jnp.dot(gamma, Wk, preferred_element_type=jnp.float32) + q)

    m = jnp.maximum(jnp.maximum(e0, e1), e2)
    a0 = jnp.exp(e0 - m)
    a1 = jnp.exp(e1 - m)
    a2 = jnp.exp(e2 - m)
    h_moment = (a0 * mu + a1 * sigma + a2 * gamma) / (a0 + a1 + a2)

    out_ref[...] = (1.0 - beta) * h_i + beta * h_moment


def kernel(input, edge_index, h0, W, W_att, lamda, alpha, l):
    x = input
    src = edge_index[0].reshape(NW, NCHUNK, B)
    dst = edge_index[1].reshape(NW, NCHUNK, B)

    theta = jnp.log(lamda / l + 1.0)
    scalars = jnp.stack([jnp.float32(theta), jnp.float32(alpha)])

    Bn = 1000
    grid = N // Bn

    h0sq, h0cub = pl.pallas_call(
        _powers_body,
        grid=(grid,),
        in_specs=[pl.BlockSpec((Bn, D), lambda i: (i, 0))],
        out_specs=[pl.BlockSpec((Bn, D), lambda i: (i, 0))] * 2,
        out_shape=[jax.ShapeDtypeStruct((N, D), jnp.float32)] * 2,
    )(h0)

    zrow = jnp.zeros((TPR, D), jnp.float32)
    ones = jnp.ones((B, D), jnp.float32)

    acc = _sc_segment_sums(src, dst, x, h0, h0sq, h0cub, zrow, ones)

    out = pl.pallas_call(
        _epilogue_body,
        grid=(grid,),
        in_specs=[
            pl.BlockSpec(memory_space=pltpu.MemorySpace.SMEM),
            pl.BlockSpec((5, NC, Bn, D), lambda i: (0, 0, i, 0)),
            pl.BlockSpec((Bn, D), lambda i: (i, 0)),
            pl.BlockSpec((D, D), lambda i: (0, 0)),
            pl.BlockSpec((2 * D, D), lambda i: (0, 0)),
        ],
        out_specs=pl.BlockSpec((Bn, D), lambda i: (i, 0)),
        out_shape=jax.ShapeDtypeStruct((N, D), jnp.float32),
    )(scalars, acc, h0, W, W_att)

    return out


# speedup vs baseline: 1.0080x; 1.0080x over previous
import functools
import math

import jax
import jax.numpy as jnp
from jax import lax
from jax.experimental import pallas as pl
from jax.experimental.pallas import tpu as pltpu
from jax.experimental.pallas import tpu_sc as plsc

N = 10000
E = 320000
D = 128
NC = 2
NS = 16
NW = NC * NS
EPW = E // NW
B = 80
NCHUNK = EPW // B
NPAD = 10112
TPR = NPAD // NS


_MESH = plsc.VectorSubcoreMesh(core_axis_name="c", subcore_axis_name="s")


def _sc_segment_sums(src, dst, x, h0, h0sq, h0cub, zrow, ones):
    @functools.partial(
        pl.kernel,
        mesh=_MESH,
        out_type=jax.ShapeDtypeStruct((5, NC, NPAD, D), jnp.float32),
        scratch_types=[
            pltpu.VMEM((NCHUNK, B), jnp.int32),
            pltpu.VMEM((B,), jnp.int32),
            pltpu.VMEM((B,), jnp.int32),
            pltpu.VMEM((B,), jnp.int32),
            pltpu.VMEM((B, D), jnp.float32),
            pltpu.VMEM((B, D), jnp.float32),
            pltpu.VMEM((B, D), jnp.float32),
            pltpu.VMEM_SHARED((NPAD, D), jnp.float32),
        ] + [pltpu.SemaphoreType.DMA] * 9,
    )
    def k(src_h, dst_h, x_h, h0_h, h0sq_h, h0cub_h, zrow_h, ones_h,
          acc_out, src_v, dst0_v, dst1_v, dst2_v, rows0_v, rows1_v,
          rows2_v, acc_sh, gsem0, gsem1, gsem2, ssem0, ssem1, ssem2,
          dsem0, dsem1, dsem2):
        core = lax.axis_index("c")
        tile = lax.axis_index("s")
        wid = core * NS + tile
        rbase = pl.multiple_of(tile * TPR, 8)

        pltpu.sync_copy(src_h.at[wid], src_v)

        rows = (rows0_v, rows1_v, rows2_v)
        dsts = (dst0_v, dst1_v, dst2_v)
        gsems = (gsem0, gsem1, gsem2)
        ssems = (ssem0, ssem1, ssem2)
        dsems = (dsem0, dsem1, dsem2)

        for p, tab in enumerate((x_h, h0_h, h0sq_h, h0cub_h, None)):
            pltpu.sync_copy(zrow_h, acc_sh.at[pl.ds(rbase, TPR)])
            if tab is None:
                pltpu.sync_copy(ones_h, rows2_v)
                pltpu.sync_copy(dst_h.at[wid], src_v)
            plsc.subcore_barrier()

            if tab is None:
                @pl.loop(0, NCHUNK + 1, step=2)
                def _chunk_deg(i):
                    for b in (0, 1):
                        c = i + b
                        cc = jnp.minimum(c, NCHUNK - 1)

                        @pl.when(c < NCHUNK)
                        def _():
                            @pl.when(c >= 2)
                            def _():
                                pltpu.make_async_copy(
                                    rows2_v, acc_sh.at[src_v.at[cc]],
                                    ssems[b]).wait()

                            pltpu.async_copy(rows2_v,
                                             acc_sh.at[src_v.at[cc]],
                                             ssems[b], add=True)

                pltpu.make_async_copy(rows2_v, acc_sh.at[src_v.at[0]],
                                      ssem0).wait()
                pltpu.make_async_copy(rows2_v, acc_sh.at[src_v.at[0]],
                                      ssem1).wait()
            else:
                pltpu.async_copy(dst_h.at[wid, 0], dst0_v, dsem0)
                pltpu.async_copy(dst_h.at[wid, 1], dst1_v, dsem1)
                pltpu.async_copy(tab.at[src_v.at[0]], rows0_v, gsem0)
                pltpu.async_copy(tab.at[src_v.at[1]], rows1_v, gsem1)

                @pl.loop(0, NCHUNK + 1, step=3)
                def _chunk(i):
                    for b in (0, 1, 2):
                        c = i + b
                        bp = (b - 1) % 3
                        cc = jnp.minimum(c, NCHUNK - 1)
                        c2 = jnp.minimum(c + 2, NCHUNK - 1)

                        @pl.when(c < NCHUNK)
                        def _():
                            pltpu.make_async_copy(dst_h.at[wid, cc],
                                                  dsts[b], dsems[b]).wait()
                            pltpu.make_async_copy(tab.at[src_v.at[cc]],
                                                  rows[b], gsems[b]).wait()
                            pltpu.async_copy(rows[b], acc_sh.at[dsts[b]],
                                             ssems[b], add=True)

                            @pl.when(c >= 1)
                            def _():
                                pltpu.make_async_copy(
                                    rows[bp], acc_sh.at[dsts[bp]],
                                    ssems[bp]).wait()

                            @pl.when(c + 2 < NCHUNK)
                            def _():
                                pltpu.async_copy(dst_h.at[wid, c2],
                                                 dsts[bp], dsems[bp])
                                pltpu.async_copy(tab.at[src_v.at[c2]],
                                                 rows[bp], gsems[bp])

                fb = (NCHUNK - 1) % 3
                pltpu.make_async_copy(rows[fb], acc_sh.at[dsts[fb]],
                                      ssems[fb]).wait()

            plsc.subcore_barrier()
            pltpu.sync_copy(acc_sh.at[pl.ds(rbase, TPR)],
                            acc_out.at[p, core, pl.ds(rbase, TPR)])

    return k(src, dst, x, h0, h0sq, h0cub, zrow, ones)


def _powers_body(h0_ref, sq_ref, cub_ref):
    h = h0_ref[...]
    s = h * h
    sq_ref[...] = s
    cub_ref[...] = s * h


def _epilogue_body(scal_ref, acc_ref, h0_ref, W_ref, Watt_ref, out_ref):
    theta = scal_ref[0]
    alpha = scal_ref[1]
    beta = 0.1

    deg = acc_ref[4, 0, :, :1] + acc_ref[4, 1, :, :1]
    invd = 1.0 / jnp.maximum(deg, 1.0)

    hx = (acc_ref[0, 0] + acc_ref[0, 1]) * invd
    hm = (acc_ref[1, 0] + acc_ref[1, 1]) * invd
    hs = (acc_ref[2, 0] + acc_ref[2, 1]) * invd
    hg = (acc_ref[3, 0] + acc_ref[3, 1]) * invd

    h_agg = (1.0 - alpha) * hx + alpha * h0_ref[...]
    hW = jnp.dot(h_agg, W_ref[...], preferred_element_type=jnp.float32)
    h_i = theta * hW + (1.0 - theta) * h_agg

    mu = hm
    s = jnp.where(hs == 0.0, 1e-16, hs)
    sigma = jnp.sqrt(s)
    g = jnp.where(hg == 0.0, 1e-16, hg)
    neg = g < 0.0
    ga = jnp.abs(g)
    gr = jnp.exp(jnp.log(ga) * (1.0 / 3.0))
    gamma = jnp.where(neg, -gr, gr)

    Wk = Watt_ref[:D, :]
    Wq = Watt_ref[D:, :]
    q = jnp.dot(h_i, Wq, preferred_element_type=jnp.float32)

    def elu(v):
        return jnp.where(v > 0.0, v, jnp.exp(jnp.minimum(v, 0.0)) - 1.0)

    e0 = elu(jnp.dot(mu, Wk, preferred_element_type=jnp.float32) + q)
    e1 = elu(jnp.dot(sigma, Wk, preferred_element_type=jnp.float32) + q)
    e2 = elu(jnp.dot(gamma, Wk, preferred_element_type=jnp.float32) + q)

    m = jnp.maximum(jnp.maximum(e0, e1), e2)
    a0 = jnp.exp(e0 - m)
    a1 = jnp.exp(e1 - m)
    a2 = jnp.exp(e2 - m)
    h_moment = (a0 * mu + a1 * sigma + a2 * gamma) / (a0 + a1 + a2)

    out_ref[...] = (1.0 - beta) * h_i + beta * h_moment


def kernel(input, edge_index, h0, W, W_att, lamda, alpha, l):
    x = input
    src = edge_index[0].reshape(NW, NCHUNK, B)
    dst = edge_index[1].reshape(NW, NCHUNK, B)

    theta = jnp.log(lamda / l + 1.0)
    scalars = jnp.stack([jnp.float32(theta), jnp.float32(alpha)])

    Bn = 1000
    grid = N // Bn

    h0sq, h0cub = pl.pallas_call(
        _powers_body,
        grid=(grid,),
        in_specs=[pl.BlockSpec((Bn, D), lambda i: (i, 0))],
        out_specs=[pl.BlockSpec((Bn, D), lambda i: (i, 0))] * 2,
        out_shape=[jax.ShapeDtypeStruct((N, D), jnp.float32)] * 2,
    )(h0)

    zrow = jnp.zeros((TPR, D), jnp.float32)
    ones = jnp.ones((B, D), jnp.float32)

    acc = _sc_segment_sums(src, dst, x, h0, h0sq, h0cub, zrow, ones)

    out = pl.pallas_call(
        _epilogue_body,
        grid=(grid,),
        in_specs=[
            pl.BlockSpec(memory_space=pltpu.MemorySpace.SMEM),
            pl.BlockSpec((5, NC, Bn, D), lambda i: (0, 0, i, 0)),
            pl.BlockSpec((Bn, D), lambda i: (i, 0)),
            pl.BlockSpec((D, D), lambda i: (0, 0)),
            pl.BlockSpec((2 * D, D), lambda i: (0, 0)),
        ],
        out_specs=pl.BlockSpec((Bn, D), lambda i: (i, 0)),
        out_shape=jax.ShapeDtypeStruct((N, D), jnp.float32),
    )(scalars, acc, h0, W, W_att)

    return out
